# Initial kernel scaffold; baseline (speedup 1.0000x reference)
import jax
import jax.numpy as jnp
from jax.experimental import pallas as pl


def kernel(x, edge_index, Wb1, Wc1, bc1, b1, Wb2, Wc2, bc2, b2, Wcls, bcls):
    raise NotImplementedError("write your pallas kernel here")


# speedup vs baseline: 16.4284x; 16.4284x over previous
import functools

import jax
import jax.numpy as jnp
from jax import lax
from jax.experimental import pallas as pl
from jax.experimental.pallas import tpu as pltpu
from jax.experimental.pallas import tpu_sc as plsc

NUM_HEADS = 8
NUM_BASES = 4
F = 64
HID = 128
NC = 2
NS = 16
NW = NC * NS
CHUNK = 125
DEGW = 8
BN = 1000


def _sc_mesh():
    return plsc.VectorSubcoreMesh(
        core_axis_name="c", subcore_axis_name="s", num_cores=NC,
        num_subcores=NS)


_SC_PARAMS = pltpu.CompilerParams(use_tc_tiling_on_sc=False)
_SC_PARAMS_NOLAYOUT = pltpu.CompilerParams(
    use_tc_tiling_on_sc=False, needs_layout_passes=False)


def _degree_partials(col2d, ones_in, zeros_in, npad):
    rows2d = col2d.shape[0]
    rpt = rows2d // NW
    npt = npad // NS

    @functools.partial(
        pl.kernel,
        out_type=jax.ShapeDtypeStruct((NC, npad, DEGW), jnp.float32),
        mesh=_sc_mesh(),
        compiler_params=_SC_PARAMS,
        scratch_types=[
            pltpu.VMEM((rpt, CHUNK), jnp.int32),
            pltpu.VMEM((CHUNK, DEGW), jnp.float32),
            pltpu.VMEM_SHARED((npad, DEGW), jnp.float32),
        ],
    )
    def k(col_hbm, ones_hbm, zeros_hbm, out_hbm, ci_v, ones_v, acc_sh):
        c = lax.axis_index("c")
        s = lax.axis_index("s")
        w = s * NC + c
        pltpu.sync_copy(zeros_hbm, acc_sh.at[pl.ds(s * npt, npt)])
        pltpu.sync_copy(ones_hbm, ones_v)
        pltpu.sync_copy(col_hbm.at[pl.ds(w * rpt, rpt)], ci_v)
        plsc.subcore_barrier()

        def body(j, carry):
            pltpu.sync_copy(ones_v, acc_sh.at[ci_v.at[j]], add=True)
            return carry

        lax.fori_loop(0, rpt, body, 0)
        plsc.subcore_barrier()
        pltpu.sync_copy(acc_sh.at[pl.ds(s * npt, npt)],
                        out_hbm.at[c, pl.ds(s * npt, npt)])

    return k(col2d, ones_in, zeros_in)


def _scatter_partials(sb, row2d, col2d, zeros_in, npad):
    rows2d = row2d.shape[0]
    rpt = rows2d // NW
    npt = npad // NS

    @functools.partial(
        pl.kernel,
        out_type=jax.ShapeDtypeStruct((NC, npad, F), jnp.float32),
        mesh=_sc_mesh(),
        compiler_params=_SC_PARAMS,
        scratch_types=[
            pltpu.VMEM((rpt, CHUNK), jnp.int32),
            pltpu.VMEM((rpt, CHUNK), jnp.int32),
            pltpu.VMEM((CHUNK, F), jnp.float32),
            pltpu.VMEM((CHUNK, F), jnp.float32),
            pltpu.VMEM_SHARED((npad, F), jnp.float32),
            pltpu.SemaphoreType.DMA,
            pltpu.SemaphoreType.DMA,
        ],
    )
    def k(sb_hbm, row_hbm, col_hbm, zeros_hbm, out_hbm,
          ri_v, ci_v, gb0, gb1, acc_sh, sem0, sem1):
        c = lax.axis_index("c")
        s = lax.axis_index("s")
        w = s * NC + c
        pltpu.sync_copy(zeros_hbm, acc_sh.at[pl.ds(s * npt, npt)])
        pltpu.sync_copy(row_hbm.at[pl.ds(w * rpt, rpt)], ri_v)
        pltpu.sync_copy(col_hbm.at[pl.ds(w * rpt, rpt)], ci_v)
        plsc.subcore_barrier()

        pltpu.async_copy(sb_hbm.at[ri_v.at[0]], gb0, sem0)

        def body(i, carry):
            j0 = 2 * i
            j1 = 2 * i + 1
            pltpu.async_copy(sb_hbm.at[ri_v.at[j1]], gb1, sem1)
            pltpu.make_async_copy(sb_hbm.at[ri_v.at[j0]], gb0, sem0).wait()
            pltpu.sync_copy(gb0, acc_sh.at[ci_v.at[j0]], add=True)

            @pl.when(i < rpt // 2 - 1)
            def _():
                pltpu.async_copy(sb_hbm.at[ri_v.at[j0 + 2]], gb0, sem0)

            pltpu.make_async_copy(sb_hbm.at[ri_v.at[j1]], gb1, sem1).wait()
            pltpu.sync_copy(gb1, acc_sh.at[ci_v.at[j1]], add=True)
            return carry

        lax.fori_loop(0, rpt // 2, body, 0)
        plsc.subcore_barrier()
        pltpu.sync_copy(acc_sh.at[pl.ds(s * npt, npt)],
                        out_hbm.at[c, pl.ds(s * npt, npt)])

    return k(sb, row2d, col2d, zeros_in)


def _edge_outputs(tflat, row, col, n, e):
    ept = e // NW
    groups = ept // 16

    @functools.partial(
        pl.kernel,
        out_type=jax.ShapeDtypeStruct((2 * e,), jnp.float32),
        mesh=_sc_mesh(),
        compiler_params=_SC_PARAMS_NOLAYOUT,
        scratch_types=[
            pltpu.VMEM((4 * n,), jnp.float32),
            pltpu.VMEM((ept,), jnp.int32),
            pltpu.VMEM((ept,), jnp.int32),
            pltpu.VMEM((2 * ept,), jnp.float32),
        ],
    )
    def k(t_hbm, row_hbm, col_hbm, out_hbm, tbuf, ri_v, ci_v, obuf):
        c = lax.axis_index("c")
        s = lax.axis_index("s")
        w = s * NC + c
        pltpu.sync_copy(t_hbm, tbuf)
        pltpu.sync_copy(row_hbm.at[pl.ds(w * ept, ept)], ri_v)
        pltpu.sync_copy(col_hbm.at[pl.ds(w * ept, ept)], ci_v)
        io2 = 2 * lax.iota(jnp.int32, 16)

        def body(g, carry):
            ri = ri_v[pl.ds(g * 16, 16)] * 4
            ci = ci_v[pl.ds(g * 16, 16)] * 4
            a0 = plsc.load_gather(tbuf, [ri])
            a1 = plsc.load_gather(tbuf, [ri + 1])
            b0 = plsc.load_gather(tbuf, [ci + 2])
            b1 = plsc.load_gather(tbuf, [ci + 3])
            base = g * 32 + io2
            plsc.store_scatter(obuf, [base], a0 + b0)
            plsc.store_scatter(obuf, [base + 1], a1 + b1)
            return carry

        lax.fori_loop(0, groups, body, 0)
        pltpu.sync_copy(obuf, out_hbm.at[pl.ds(w * 2 * ept, 2 * ept)])

    return k(tflat, row, col)


def _einsum_combine(wts, agg):
    fh = HID // NUM_HEADS
    cols = []
    for h in range(NUM_HEADS):
        acc = wts[:, 4 * h:4 * h + 1] * agg[:, 0:fh]
        for b in range(1, NUM_BASES):
            acc = acc + wts[:, 4 * h + b:4 * h + b + 1] * agg[:, fh * b:fh * (b + 1)]
        cols.append(acc)
    return jnp.concatenate(cols, axis=1)


def _tc_stage1(degp, x, Wb1, Wc1, bc1, n):
    def body(degp_ref, x_ref, wb_ref, wc_ref, bc_ref,
             sb_ref, w1_ref, dis_ref):
        deg = degp_ref[0] + degp_ref[1]
        dis = 1.0 / jnp.sqrt(deg[:, 0:1] + 1.0)
        dis_ref[...] = jnp.broadcast_to(dis, (BN, DEGW))
        xb = x_ref[...]
        bases = jnp.dot(xb, wb_ref[...], preferred_element_type=jnp.float32)
        sb_ref[...] = bases * dis
        w1_ref[...] = jnp.dot(xb, wc_ref[...],
                              preferred_element_type=jnp.float32) + bc_ref[...]

    grid = n // BN
    return pl.pallas_call(
        body,
        grid=(grid,),
        in_specs=[
            pl.BlockSpec((NC, BN, DEGW), lambda i: (0, i, 0)),
            pl.BlockSpec((BN, HID), lambda i: (i, 0)),
            pl.BlockSpec((HID, F), lambda i: (0, 0)),
            pl.BlockSpec((HID, 32), lambda i: (0, 0)),
            pl.BlockSpec((1, 32), lambda i: (0, 0)),
        ],
        out_specs=[
            pl.BlockSpec((BN, F), lambda i: (i, 0)),
            pl.BlockSpec((BN, 32), lambda i: (i, 0)),
            pl.BlockSpec((BN, DEGW), lambda i: (i, 0)),
        ],
        out_shape=[
            jax.ShapeDtypeStruct((n, F), jnp.float32),
            jax.ShapeDtypeStruct((n, 32), jnp.float32),
            jax.ShapeDtypeStruct((n, DEGW), jnp.float32),
        ],
    )(degp, x, Wb1, Wc1, bc1)


def _tc_stage2(scatp, sb1, dis, w1, b1, Wb2, Wc2, bc2, n):
    def body(scat_ref, sb_ref, dis_ref, w1_ref, b1_ref, wb_ref, wc_ref,
             bc_ref, sb2_ref, w2_ref):
        dis = dis_ref[...][:, 0:1]
        agg = (scat_ref[0] + scat_ref[1] + sb_ref[...]) * dis
        h1 = _einsum_combine(w1_ref[...], agg) + b1_ref[...]
        h1 = jnp.maximum(h1, 0.0)
        bases2 = jnp.dot(h1, wb_ref[...], preferred_element_type=jnp.float32)
        sb2_ref[...] = bases2 * dis
        w2_ref[...] = jnp.dot(h1, wc_ref[...],
                              preferred_element_type=jnp.float32) + bc_ref[...]

    grid = n // BN
    return pl.pallas_call(
        body,
        grid=(grid,),
        in_specs=[
            pl.BlockSpec((NC, BN, F), lambda i: (0, i, 0)),
            pl.BlockSpec((BN, F), lambda i: (i, 0)),
            pl.BlockSpec((BN, DEGW), lambda i: (i, 0)),
            pl.BlockSpec((BN, 32), lambda i: (i, 0)),
            pl.BlockSpec((1, HID), lambda i: (0, 0)),
            pl.BlockSpec((HID, F), lambda i: (0, 0)),
            pl.BlockSpec((HID, 32), lambda i: (0, 0)),
            pl.BlockSpec((1, 32), lambda i: (0, 0)),
        ],
        out_specs=[
            pl.BlockSpec((BN, F), lambda i: (i, 0)),
            pl.BlockSpec((BN, 32), lambda i: (i, 0)),
        ],
        out_shape=[
            jax.ShapeDtypeStruct((n, F), jnp.float32),
            jax.ShapeDtypeStruct((n, 32), jnp.float32),
        ],
    )(scatp, sb1, dis, w1, b1, Wb2, Wc2, bc2)


def _tc_stage3(scatp, sb2, dis, w2, b2, Wt, bt, n):
    def body(scat_ref, sb_ref, dis_ref, w2_ref, b2_ref, wt_ref, bt_ref,
             t_ref):
        dis = dis_ref[...][:, 0:1]
        agg = (scat_ref[0] + scat_ref[1] + sb_ref[...]) * dis
        h2 = _einsum_combine(w2_ref[...], agg) + b2_ref[...]
        t_ref[...] = jnp.dot(h2, wt_ref[...],
                             preferred_element_type=jnp.float32) + bt_ref[...]

    grid = n // BN
    return pl.pallas_call(
        body,
        grid=(grid,),
        in_specs=[
            pl.BlockSpec((NC, BN, F), lambda i: (0, i, 0)),
            pl.BlockSpec((BN, F), lambda i: (i, 0)),
            pl.BlockSpec((BN, DEGW), lambda i: (i, 0)),
            pl.BlockSpec((BN, 32), lambda i: (i, 0)),
            pl.BlockSpec((1, HID), lambda i: (0, 0)),
            pl.BlockSpec((HID, 4), lambda i: (0, 0)),
            pl.BlockSpec((1, 4), lambda i: (0, 0)),
        ],
        out_specs=pl.BlockSpec((BN, 4), lambda i: (i, 0)),
        out_shape=jax.ShapeDtypeStruct((n, 4), jnp.float32),
    )(scatp, sb2, dis, w2, b2, Wt, bt)


def kernel(x, edge_index, Wb1, Wc1, bc1, b1, Wb2, Wc2, bc2, b2, Wcls, bcls):
    n = x.shape[0]
    e = edge_index.shape[1]
    row = edge_index[0]
    col = edge_index[1]
    row2d = row.reshape(e // CHUNK, CHUNK)
    col2d = col.reshape(e // CHUNK, CHUNK)

    npad = -(-n // (NS * 8)) * (NS * 8)
    npt = npad // NS
    ones_in = jnp.ones((CHUNK, DEGW), jnp.float32)
    zeros_deg = jnp.zeros((npt, DEGW), jnp.float32)
    zeros_f = jnp.zeros((npt, F), jnp.float32)
    bc1r = bc1.reshape(1, -1)
    bc2r = bc2.reshape(1, -1)
    b1r = b1.reshape(1, -1)
    b2r = b2.reshape(1, -1)
    Wt = jnp.concatenate([Wcls[:HID], Wcls[HID:]], axis=1)
    bt = jnp.concatenate([bcls, jnp.zeros_like(bcls)]).reshape(1, 4)

    degp = _degree_partials(col2d, ones_in, zeros_deg, npad)
    sb1, w1, dis = _tc_stage1(degp, x, Wb1, Wc1, bc1r, n)
    scat1 = _scatter_partials(sb1, row2d, col2d, zeros_f, npad)
    sb2, w2 = _tc_stage2(scat1, sb1, dis, w1, b1r, Wb2, Wc2, bc2r, n)
    scat2 = _scatter_partials(sb2, row2d, col2d, zeros_f, npad)
    tmat = _tc_stage3(scat2, sb2, dis, w2, b2r, Wt, bt, n)

    out2 = _edge_outputs(tmat.reshape(-1), row, col, n, e)
    return out2.reshape(e, 2)


# speedup vs baseline: 50.6340x; 3.0821x over previous
import functools

import numpy as np

import jax
import jax.numpy as jnp
from jax import lax
from jax.experimental import pallas as pl
from jax.experimental.pallas import tpu as pltpu
from jax.experimental.pallas import tpu_sc as plsc

NUM_HEADS = 8
NUM_BASES = 4
F = 64
HID = 128
NC = 2
NS = 16
NW = NC * NS
CHUNK = 125
SCHUNK = 250
DEGW = 8
BN = 2000


def _sc_mesh():
    return plsc.VectorSubcoreMesh(
        core_axis_name="c", subcore_axis_name="s", num_cores=NC,
        num_subcores=NS)


_SC_PARAMS = pltpu.CompilerParams(use_tc_tiling_on_sc=False)
_SC_PARAMS_NOLAYOUT = pltpu.CompilerParams(
    use_tc_tiling_on_sc=False, needs_layout_passes=False)


def _degree_partials(ei3, ones_in, zeros_in, npad):
    eb = ei3.shape[0]
    base_b = eb // NW
    extra = eb - base_b * NW
    npt = npad // NS

    @functools.partial(
        pl.kernel,
        out_type=jax.ShapeDtypeStruct((npad, 128), jnp.float32),
        mesh=_sc_mesh(),
        compiler_params=_SC_PARAMS,
        scratch_types=[
            pltpu.VMEM((base_b + 1, 2, 128), jnp.int32),
            pltpu.VMEM((128, DEGW), jnp.float32),
            pltpu.VMEM_SHARED((npad, DEGW), jnp.float32),
        ],
    )
    def k(ei_hbm, ones_hbm, zeros_hbm, out_hbm, slab, ones_v, acc_sh):
        c = lax.axis_index("c")
        s = lax.axis_index("s")
        w = s * NC + c
        pltpu.sync_copy(zeros_hbm, acc_sh.at[pl.ds(s * npt, npt)])
        pltpu.sync_copy(ones_hbm, ones_v)
        pltpu.sync_copy(ei_hbm.at[pl.ds(w * base_b, base_b)],
                        slab.at[pl.ds(0, base_b)])

        @pl.when(w < extra)
        def _():
            pltpu.sync_copy(ei_hbm.at[pl.ds(eb - extra + w, 1)],
                            slab.at[pl.ds(base_b, 1)])

        plsc.subcore_barrier()

        def body(j, carry):
            pltpu.sync_copy(ones_v, acc_sh.at[slab.at[j, 1]], add=True)
            return carry

        lax.fori_loop(0, base_b, body, 0)

        @pl.when(w < extra)
        def _():
            pltpu.sync_copy(ones_v, acc_sh.at[slab.at[base_b, 1]], add=True)

        plsc.subcore_barrier()
        pltpu.sync_copy(acc_sh.at[pl.ds(s * npt, npt)],
                        out_hbm.at[pl.ds(s * npt, npt), pl.ds(DEGW * c, DEGW)])

    return k(ei3, ones_in, zeros_in)


def _scatter_partials(sb, row2d, col2d, zeros_in, npad):
    rows2d, chunk = row2d.shape
    rpt = rows2d // NW
    npt = npad // NS

    @functools.partial(
        pl.kernel,
        out_type=jax.ShapeDtypeStruct((npad, NC * F), jnp.float32),
        mesh=_sc_mesh(),
        compiler_params=_SC_PARAMS,
        scratch_types=[
            pltpu.VMEM((rpt, chunk), jnp.int32),
            pltpu.VMEM((rpt, chunk), jnp.int32),
            pltpu.VMEM((chunk, F), jnp.float32),
            pltpu.VMEM((chunk, F), jnp.float32),
            pltpu.VMEM_SHARED((npad, F), jnp.float32),
            pltpu.SemaphoreType.DMA,
            pltpu.SemaphoreType.DMA,
        ],
    )
    def k(sb_hbm, row_hbm, col_hbm, zeros_hbm, out_hbm,
          ri_v, ci_v, gb0, gb1, acc_sh, sem0, sem1):
        c = lax.axis_index("c")
        s = lax.axis_index("s")
        w = s * NC + c
        pltpu.sync_copy(zeros_hbm, acc_sh.at[pl.ds(s * npt, npt)])
        pltpu.sync_copy(row_hbm.at[pl.ds(w * rpt, rpt)], ri_v)
        pltpu.sync_copy(col_hbm.at[pl.ds(w * rpt, rpt)], ci_v)
        plsc.subcore_barrier()

        pltpu.async_copy(sb_hbm.at[ri_v.at[0]], gb0, sem0)

        def body(i, carry):
            j0 = 2 * i
            j1 = 2 * i + 1
            pltpu.async_copy(sb_hbm.at[ri_v.at[j1]], gb1, sem1)
            pltpu.make_async_copy(sb_hbm.at[ri_v.at[j0]], gb0, sem0).wait()
            pltpu.sync_copy(gb0, acc_sh.at[ci_v.at[j0]], add=True)

            @pl.when(i < rpt // 2 - 1)
            def _():
                pltpu.async_copy(sb_hbm.at[ri_v.at[j0 + 2]], gb0, sem0)

            pltpu.make_async_copy(sb_hbm.at[ri_v.at[j1]], gb1, sem1).wait()
            pltpu.sync_copy(gb1, acc_sh.at[ci_v.at[j1]], add=True)
            return carry

        lax.fori_loop(0, rpt // 2, body, 0)
        plsc.subcore_barrier()
        pltpu.sync_copy(acc_sh.at[pl.ds(s * npt, npt)],
                        out_hbm.at[pl.ds(s * npt, npt), pl.ds(F * c, F)])

    return k(sb, row2d, col2d, zeros_in)


def _edge_outputs(tflat, ei3, n, e):
    eb = ei3.shape[0]
    base_b = eb // NW
    extra = eb - base_b * NW

    @functools.partial(
        pl.kernel,
        out_type=jax.ShapeDtypeStruct((eb, 2, 128), jnp.float32),
        mesh=_sc_mesh(),
        compiler_params=_SC_PARAMS_NOLAYOUT,
        scratch_types=[
            pltpu.VMEM((4 * n,), jnp.float32),
            pltpu.VMEM((base_b + 1, 2, 128), jnp.int32),
            pltpu.VMEM((base_b + 1, 2, 128), jnp.float32),
        ],
    )
    def k(t_hbm, ei_hbm, out_hbm, tbuf, slab, obuf):
        c = lax.axis_index("c")
        s = lax.axis_index("s")
        w = s * NC + c
        pltpu.sync_copy(t_hbm, tbuf)
        pltpu.sync_copy(ei_hbm.at[pl.ds(w * base_b, base_b)],
                        slab.at[pl.ds(0, base_b)])

        @pl.when(w < extra)
        def _():
            pltpu.sync_copy(ei_hbm.at[pl.ds(eb - extra + w, 1)],
                            slab.at[pl.ds(base_b, 1)])

        def do_block(j):
            for l in range(8):
                ri = slab[j, 0, pl.ds(l * 16, 16)] * 4
                ci = slab[j, 1, pl.ds(l * 16, 16)] * 4
                a0 = plsc.load_gather(tbuf, [ri])
                a1 = plsc.load_gather(tbuf, [ri + 1])
                b0 = plsc.load_gather(tbuf, [ci + 2])
                b1 = plsc.load_gather(tbuf, [ci + 3])
                obuf[j, 0, pl.ds(l * 16, 16)] = a0 + b0
                obuf[j, 1, pl.ds(l * 16, 16)] = a1 + b1

        def body(j, carry):
            do_block(j)
            return carry

        lax.fori_loop(0, base_b, body, 0)

        @pl.when(w < extra)
        def _():
            do_block(base_b)

        pltpu.sync_copy(obuf.at[pl.ds(0, base_b)],
                        out_hbm.at[pl.ds(w * base_b, base_b)])

        @pl.when(w < extra)
        def _():
            pltpu.sync_copy(obuf.at[pl.ds(base_b, 1)],
                            out_hbm.at[pl.ds(eb - extra + w, 1)])

    return k(tflat, ei3)


def _einsum_combine(wts, agg, e_ref, f_ref):
    w2 = jnp.dot(wts, e_ref[...], preferred_element_type=jnp.float32)
    a2 = jnp.dot(agg, f_ref[...], preferred_element_type=jnp.float32)
    p = w2 * a2
    return (p[:, 0:128] + p[:, 128:256]) + (p[:, 256:384] + p[:, 384:512])


def _einsum_selectors():
    li = np.arange(NUM_BASES * HID)
    l_b = li // HID
    l_h = (li % HID) // (HID // NUM_HEADS)
    l_f = li % (HID // NUM_HEADS)
    e = (np.arange(NUM_HEADS * NUM_BASES)[:, None]
         == (NUM_BASES * l_h + l_b)[None, :]).astype(np.float32)
    f = (np.arange(F)[:, None]
         == ((HID // NUM_HEADS) * l_b + l_f)[None, :]).astype(np.float32)
    return jnp.asarray(e), jnp.asarray(f)


def _tc_stage1(degp, x, Wb1, Wc1, bc1, n):
    def body(degp_ref, x_ref, wb_ref, wc_ref, bc_ref,
             sb_ref, w1_ref, dis_ref):
        degp = degp_ref[...]
        deg = degp[:, 0:1] + degp[:, DEGW:DEGW + 1]
        dis = 1.0 / jnp.sqrt(deg + 1.0)
        dis_ref[...] = jnp.broadcast_to(dis, (BN, DEGW))
        xb = x_ref[...]
        bases = jnp.dot(xb, wb_ref[...], preferred_element_type=jnp.float32)
        sb_ref[...] = bases * dis
        w1_ref[...] = jnp.dot(xb, wc_ref[...],
                              preferred_element_type=jnp.float32) + bc_ref[...]

    grid = n // BN
    return pl.pallas_call(
        body,
        grid=(grid,),
        in_specs=[
            pl.BlockSpec((BN, 128), lambda i: (i, 0)),
            pl.BlockSpec((BN, HID), lambda i: (i, 0)),
            pl.BlockSpec((HID, F), lambda i: (0, 0)),
            pl.BlockSpec((HID, 32), lambda i: (0, 0)),
            pl.BlockSpec((1, 32), lambda i: (0, 0)),
        ],
        out_specs=[
            pl.BlockSpec((BN, F), lambda i: (i, 0)),
            pl.BlockSpec((BN, 32), lambda i: (i, 0)),
            pl.BlockSpec((BN, DEGW), lambda i: (i, 0)),
        ],
        out_shape=[
            jax.ShapeDtypeStruct((n, F), jnp.float32),
            jax.ShapeDtypeStruct((n, 32), jnp.float32),
            jax.ShapeDtypeStruct((n, DEGW), jnp.float32),
        ],
    )(degp, x, Wb1, Wc1, bc1)


def _tc_stage2(scatp, sb1, dis, w1, b1, Wb2, Wc2, bc2, emat, fmat, n):
    def body(scat_ref, sb_ref, dis_ref, w1_ref, b1_ref, wb_ref, wc_ref,
             bc_ref, e_ref, f_ref, sb2_ref, w2_ref):
        dis = dis_ref[...][:, 0:1]
        scat = scat_ref[...]
        agg = (scat[:, 0:F] + scat[:, F:2 * F] + sb_ref[...]) * dis
        h1 = _einsum_combine(w1_ref[...], agg, e_ref, f_ref) + b1_ref[...]
        h1 = jnp.maximum(h1, 0.0)
        bases2 = jnp.dot(h1, wb_ref[...], preferred_element_type=jnp.float32)
        sb2_ref[...] = bases2 * dis
        w2_ref[...] = jnp.dot(h1, wc_ref[...],
                              preferred_element_type=jnp.float32) + bc_ref[...]

    grid = n // BN
    return pl.pallas_call(
        body,
        grid=(grid,),
        in_specs=[
            pl.BlockSpec((BN, NC * F), lambda i: (i, 0)),
            pl.BlockSpec((BN, F), lambda i: (i, 0)),
            pl.BlockSpec((BN, DEGW), lambda i: (i, 0)),
            pl.BlockSpec((BN, 32), lambda i: (i, 0)),
            pl.BlockSpec((1, HID), lambda i: (0, 0)),
            pl.BlockSpec((HID, F), lambda i: (0, 0)),
            pl.BlockSpec((HID, 32), lambda i: (0, 0)),
            pl.BlockSpec((1, 32), lambda i: (0, 0)),
            pl.BlockSpec((NUM_HEADS * NUM_BASES, NUM_BASES * HID),
                         lambda i: (0, 0)),
            pl.BlockSpec((F, NUM_BASES * HID), lambda i: (0, 0)),
        ],
        out_specs=[
            pl.BlockSpec((BN, F), lambda i: (i, 0)),
            pl.BlockSpec((BN, 32), lambda i: (i, 0)),
        ],
        out_shape=[
            jax.ShapeDtypeStruct((n, F), jnp.float32),
            jax.ShapeDtypeStruct((n, 32), jnp.float32),
        ],
    )(scatp, sb1, dis, w1, b1, Wb2, Wc2, bc2, emat, fmat)


def _tc_stage3(scatp, sb2, dis, w2, b2, Wt, bt, emat, fmat, n):
    def body(scat_ref, sb_ref, dis_ref, w2_ref, b2_ref, wt_ref, bt_ref,
             e_ref, f_ref, t_ref):
        dis = dis_ref[...][:, 0:1]
        scat = scat_ref[...]
        agg = (scat[:, 0:F] + scat[:, F:2 * F] + sb_ref[...]) * dis
        h2 = _einsum_combine(w2_ref[...], agg, e_ref, f_ref) + b2_ref[...]
        t_ref[...] = jnp.dot(h2, wt_ref[...],
                             preferred_element_type=jnp.float32) + bt_ref[...]

    grid = n // BN
    return pl.pallas_call(
        body,
        grid=(grid,),
        in_specs=[
            pl.BlockSpec((BN, NC * F), lambda i: (i, 0)),
            pl.BlockSpec((BN, F), lambda i: (i, 0)),
            pl.BlockSpec((BN, DEGW), lambda i: (i, 0)),
            pl.BlockSpec((BN, 32), lambda i: (i, 0)),
            pl.BlockSpec((1, HID), lambda i: (0, 0)),
            pl.BlockSpec((HID, 4), lambda i: (0, 0)),
            pl.BlockSpec((1, 4), lambda i: (0, 0)),
            pl.BlockSpec((NUM_HEADS * NUM_BASES, NUM_BASES * HID),
                         lambda i: (0, 0)),
            pl.BlockSpec((F, NUM_BASES * HID), lambda i: (0, 0)),
        ],
        out_specs=pl.BlockSpec((BN, 4), lambda i: (i, 0)),
        out_shape=jax.ShapeDtypeStruct((n, 4), jnp.float32),
    )(scatp, sb2, dis, w2, b2, Wt, bt, emat, fmat)


def kernel(x, edge_index, Wb1, Wc1, bc1, b1, Wb2, Wc2, bc2, b2, Wcls, bcls):
    n = x.shape[0]
    e = edge_index.shape[1]
    ei3 = jnp.transpose(edge_index.reshape(2, e // 128, 128), (1, 0, 2))
    row = edge_index[0]
    col = edge_index[1]
    row_sc = row.reshape(e // SCHUNK, SCHUNK)
    col_sc = col.reshape(e // SCHUNK, SCHUNK)

    npad = -(-n // (NS * 8)) * (NS * 8)
    npt = npad // NS
    ones_in = jnp.ones((128, DEGW), jnp.float32)
    zeros_deg = jnp.zeros((npt, DEGW), jnp.float32)
    zeros_f = jnp.zeros((npt, F), jnp.float32)
    bc1r = bc1.reshape(1, -1)
    bc2r = bc2.reshape(1, -1)
    b1r = b1.reshape(1, -1)
    b2r = b2.reshape(1, -1)
    Wt = jnp.concatenate([Wcls[:HID], Wcls[HID:]], axis=1)
    bt = jnp.concatenate([bcls, jnp.zeros_like(bcls)]).reshape(1, 4)

    degp = _degree_partials(ei3, ones_in, zeros_deg, npad)
    sb1, w1, dis = _tc_stage1(degp, x, Wb1, Wc1, bc1r, n)
    scat1 = _scatter_partials(sb1, row_sc, col_sc, zeros_f, npad)
    emat, fmat = _einsum_selectors()
    sb2, w2 = _tc_stage2(scat1, sb1, dis, w1, b1r, Wb2, Wc2, bc2r,
                         emat, fmat, n)
    scat2 = _scatter_partials(sb2, row_sc, col_sc, zeros_f, npad)
    tmat = _tc_stage3(scat2, sb2, dis, w2, b2r, Wt, bt, emat, fmat, n)

    out3 = _edge_outputs(tmat.reshape(-1), ei3, n, e)
    return jnp.swapaxes(out3, 0, 1).reshape(2, e).T
